# Initial kernel scaffold; baseline (speedup 1.0000x reference)
import jax
import jax.numpy as jnp
from jax.experimental import pallas as pl


def kernel(x, edge_index, W1, b1, g1, be1, W2, b2, g2, be2, W3, b3):
    raise NotImplementedError("write your pallas kernel here")


# speedup vs baseline: 7.7869x; 7.7869x over previous
import functools

import jax
import jax.numpy as jnp
from jax import lax
from jax.experimental import pallas as pl
from jax.experimental.pallas import tpu as pltpu
from jax.experimental.pallas import tpu_sc as plsc

_N = 10000
_E = 320000
_D_IN = 128
_D_H = 128
_D_OUT = 40

_NC = 2
_NS = 16
_NW = _NC * _NS
_K = 128
_CH = 80
_E_PAD = _NW * _CH * _K
_NP = 10112
_RPW = _NP // _NS


def _make_propagate(D):
    mesh = plsc.VectorSubcoreMesh(core_axis_name="c", subcore_axis_name="s")

    @functools.partial(
        pl.kernel,
        out_type=jax.ShapeDtypeStruct((_NC, _NP, D), jnp.float32),
        mesh=mesh,
        scratch_types=[
            pltpu.VMEM((_CH, _K), jnp.int32),
            pltpu.VMEM((_CH, _K), jnp.int32),
            pltpu.VMEM((_K, D), jnp.float32),
            pltpu.VMEM_SHARED((_NP, D), jnp.float32),
        ],
        compiler_params=pltpu.CompilerParams(use_tc_tiling_on_sc=False),
    )
    def kern(u_hbm, src_hbm, dst_hbm, zeros_hbm, out_hbm, src_v, dst_v, rows_v, acc):
        c = lax.axis_index("c")
        s = lax.axis_index("s")
        w = c * _NS + s
        pltpu.sync_copy(src_hbm.at[w], src_v)
        pltpu.sync_copy(dst_hbm.at[w], dst_v)
        r0 = s * _RPW
        pltpu.sync_copy(zeros_hbm.at[pl.ds(r0, _RPW)], acc.at[pl.ds(r0, _RPW)])
        plsc.subcore_barrier()

        def body(i, carry):
            pltpu.sync_copy(u_hbm.at[src_v.at[i]], rows_v)
            pltpu.sync_copy(rows_v, acc.at[dst_v.at[i]], add=True)
            return carry

        lax.fori_loop(0, _CH, body, 0)
        plsc.subcore_barrier()
        pltpu.sync_copy(acc.at[pl.ds(r0, _RPW)], out_hbm.at[c, pl.ds(r0, _RPW)])

    return kern


_prop16 = _make_propagate(16)
_prop128 = _make_propagate(128)
_prop48 = _make_propagate(48)


def _pre_body(x_ref, w_ref, degt_ref, h_ref, u_ref, dinv_ref):
    d = degt_ref[:, 0:1] + degt_ref[:, 1:2] + 1.0
    dinv = lax.rsqrt(d)
    h = jnp.dot(x_ref[...], w_ref[...], preferred_element_type=jnp.float32)
    h_ref[...] = h
    u_ref[...] = h * dinv
    dinv_ref[...] = dinv


@jax.jit
def _pre(x, W, degt):
    return pl.pallas_call(
        _pre_body,
        out_shape=(
            jax.ShapeDtypeStruct((_N, _D_H), jnp.float32),
            jax.ShapeDtypeStruct((_N, _D_H), jnp.float32),
            jax.ShapeDtypeStruct((_N, 1), jnp.float32),
        ),
    )(x, W, degt)


def _mid_body(s_ref, h_ref, dinv_ref, b_ref, g_ref, be_ref, w_ref, hn_ref, un_ref):
    dinv = dinv_ref[...]
    agg = (dinv * (s_ref[0, :_N, :] + s_ref[1, :_N, :])
           + (dinv * dinv) * h_ref[...] + b_ref[...])
    m = jnp.mean(agg, axis=0, keepdims=True)
    v = jnp.mean((agg - m) ** 2, axis=0, keepdims=True)
    z = jnp.maximum(g_ref[...] * (agg - m) * lax.rsqrt(v + 1e-5) + be_ref[...], 0.0)
    hn = jnp.dot(z, w_ref[...], preferred_element_type=jnp.float32)
    hn_ref[...] = hn
    un_ref[...] = hn * dinv


@functools.partial(jax.jit, static_argnames=("dn",))
def _mid(S, h, dinv, b, g, be, Wn, dn):
    return pl.pallas_call(
        _mid_body,
        out_shape=(
            jax.ShapeDtypeStruct((_N, dn), jnp.float32),
            jax.ShapeDtypeStruct((_N, dn), jnp.float32),
        ),
    )(S, h, dinv, b.reshape(1, -1), g.reshape(1, -1), be.reshape(1, -1), Wn)


def _post_body(s_ref, h_ref, dinv_ref, b_ref, out_ref):
    dinv = dinv_ref[...]
    agg = (dinv * (s_ref[0, :_N, :_D_OUT] + s_ref[1, :_N, :_D_OUT])
           + (dinv * dinv) * h_ref[...] + b_ref[...])
    m = jnp.max(agg, axis=-1, keepdims=True)
    e = agg - m
    lse = jnp.log(jnp.sum(jnp.exp(e), axis=-1, keepdims=True))
    out_ref[...] = e - lse


@jax.jit
def _post(S, h, dinv, b):
    return pl.pallas_call(
        _post_body,
        out_shape=jax.ShapeDtypeStruct((_N, _D_OUT), jnp.float32),
    )(S, h, dinv, b.reshape(1, -1))


def _pad_rows(u, width):
    out = jnp.zeros((_NP, width), jnp.float32)
    return out.at[: u.shape[0], : u.shape[1]].set(u)


def kernel(x, edge_index, W1, b1, g1, be1, W2, b2, g2, be2, W3, b3):
    pad = _E_PAD - _E
    fill = jnp.full((pad,), _N, jnp.int32)
    src3 = jnp.concatenate([edge_index[0], fill]).reshape(_NW, _CH, _K)
    dst3 = jnp.concatenate([edge_index[1], fill]).reshape(_NW, _CH, _K)

    z128 = jnp.zeros((_NP, 128), jnp.float32)
    z48 = jnp.zeros((_NP, 48), jnp.float32)
    z16 = jnp.zeros((_NP, 16), jnp.float32)
    ones16 = jnp.ones((_NP, 16), jnp.float32)

    degp = _prop16(ones16, src3, dst3, z16)
    degt = jnp.transpose(degp[:, :_N, 0])

    h1, u1, dinv = _pre(x, W1, degt)
    S1 = _prop128(_pad_rows(u1, 128), src3, dst3, z128)
    h2, u2 = _mid(S1, h1, dinv, b1, g1, be1, W2, dn=_D_H)
    S2 = _prop128(_pad_rows(u2, 128), src3, dst3, z128)
    h3, u3 = _mid(S2, h2, dinv, b2, g2, be2, W3, dn=_D_OUT)
    S3 = _prop48(_pad_rows(u3, 48), src3, dst3, z48)
    return _post(S3, h3, dinv, b3)
